# baseline (speedup 1.0000x reference)
import jax
import jax.numpy as jnp
from jax import lax
from jax.experimental import pallas as pl
from jax.experimental.pallas import tpu as pltpu
from jax.experimental.pallas import tpu_sc as plsc

N = 10000
D = 128
E = 320000

NC = 2
NS = 16
NW = NC * NS
C = 128
NCH = 80
NH = NCH // 2
EPT = NCH * C
EPAD = NW * EPT
NPAD = 10112
RPT = NPAD // NS
NDEG = 10240
DPT = NDEG // NS

_mesh = plsc.VectorSubcoreMesh(core_axis_name="c", subcore_axis_name="s")


def _sc_deg_body(dstT_hbm, z1_hbm, degp_hbm, dst_v, ones_v, acc_sh):
    c = lax.axis_index("c")
    s = lax.axis_index("s")
    wid = c * NS + s
    pltpu.sync_copy(dstT_hbm.at[wid], dst_v)
    for k in range(C // 16):
        ones_v[pl.ds(k * 16, 16)] = jnp.ones((16,), jnp.float32)
    pltpu.sync_copy(z1_hbm.at[pl.ds(s * DPT, DPT)], acc_sh.at[pl.ds(s * DPT, DPT)])
    plsc.subcore_barrier()

    def step(j, carry):
        pltpu.sync_copy(ones_v, acc_sh.at[dst_v.at[j]], add=True)
        return carry

    lax.fori_loop(0, NCH, step, 0)
    plsc.subcore_barrier()
    pltpu.sync_copy(acc_sh.at[pl.ds(s * DPT, DPT)], degp_hbm.at[c, pl.ds(s * DPT, DPT)])


_sc_deg = pl.kernel(
    _sc_deg_body,
    out_type=jax.ShapeDtypeStruct((NC, NDEG), jnp.float32),
    mesh=_mesh,
    scratch_types=[
        pltpu.VMEM((NCH, C), jnp.int32),
        pltpu.VMEM((C,), jnp.float32),
        pltpu.VMEM_SHARED((NDEG,), jnp.float32),
    ],
)


def _sc_agg_body(h2_hbm, srcT_hbm, dstT_hbm, zrows_hbm, aggp_hbm,
                 src_v, dst_v, rows0, rows1, acc_sh, sg0, sg1, ss0, ss1):
    c = lax.axis_index("c")
    s = lax.axis_index("s")
    wid = c * NS + s
    pltpu.sync_copy(srcT_hbm.at[wid, pl.ds(0, NH)], src_v)
    pltpu.sync_copy(dstT_hbm.at[wid, pl.ds(0, NH)], dst_v)
    pltpu.sync_copy(zrows_hbm.at[pl.ds(s * RPT, RPT)], acc_sh.at[pl.ds(s * RPT, RPT)])
    plsc.subcore_barrier()

    for half in range(2):
        pltpu.async_copy(h2_hbm.at[src_v.at[0]], rows0, sg0)
        pltpu.async_copy(h2_hbm.at[src_v.at[1]], rows1, sg1)

        def step(i, carry):
            j0 = 2 * i
            j1 = j0 + 1
            j2 = jnp.minimum(j0 + 2, NH - 1)
            j3 = jnp.minimum(j1 + 2, NH - 1)
            pltpu.make_async_copy(h2_hbm.at[src_v.at[j0]], rows0, sg0).wait()
            pltpu.async_copy(rows0, acc_sh.at[dst_v.at[j0]], ss0, add=True)
            pltpu.make_async_copy(h2_hbm.at[src_v.at[j1]], rows1, sg1).wait()
            pltpu.async_copy(rows1, acc_sh.at[dst_v.at[j1]], ss1, add=True)
            pltpu.make_async_copy(rows0, acc_sh.at[dst_v.at[j0]], ss0).wait()
            pltpu.async_copy(h2_hbm.at[src_v.at[j2]], rows0, sg0)
            pltpu.make_async_copy(rows1, acc_sh.at[dst_v.at[j1]], ss1).wait()
            pltpu.async_copy(h2_hbm.at[src_v.at[j3]], rows1, sg1)
            return carry

        lax.fori_loop(0, NH // 2, step, 0)
        pltpu.make_async_copy(h2_hbm.at[src_v.at[NH - 1]], rows0, sg0).wait()
        pltpu.make_async_copy(h2_hbm.at[src_v.at[NH - 1]], rows1, sg1).wait()
        if half == 0:
            pltpu.sync_copy(srcT_hbm.at[wid, pl.ds(NH, NH)], src_v)
            pltpu.sync_copy(dstT_hbm.at[wid, pl.ds(NH, NH)], dst_v)

    plsc.subcore_barrier()
    pltpu.sync_copy(acc_sh.at[pl.ds(s * RPT, RPT)],
                    aggp_hbm.at[c, pl.ds(s * RPT, RPT)])


_sc_agg = pl.kernel(
    _sc_agg_body,
    out_type=jax.ShapeDtypeStruct((NC, NPAD, D), jnp.float32),
    mesh=_mesh,
    scratch_types=[
        pltpu.VMEM((NH, C), jnp.int32),
        pltpu.VMEM((NH, C), jnp.int32),
        pltpu.VMEM((C, D), jnp.float32),
        pltpu.VMEM((C, D), jnp.float32),
        pltpu.VMEM_SHARED((NPAD, D), jnp.float32),
        pltpu.SemaphoreType.DMA,
        pltpu.SemaphoreType.DMA,
        pltpu.SemaphoreType.DMA,
        pltpu.SemaphoreType.DMA,
    ],
)


def _tc_rsqrt_body(degp_ref, dinv_ref):
    deg = degp_ref[0] + degp_ref[1] + 1.0
    dinv_ref[...] = lax.rsqrt(deg)


def _tc_rsqrt(degp):
    return pl.pallas_call(
        _tc_rsqrt_body,
        out_shape=jax.ShapeDtypeStruct((NDEG // D, D), jnp.float32),
    )(degp)


def _tc_pre_body(x_ref, w_ref, dinv_ref, h2_ref):
    hw = lax.dot_general(x_ref[...], w_ref[...], (((1,), (1,)), ((), ())),
                         preferred_element_type=jnp.float32)
    h2_ref[...] = hw * dinv_ref[...]


def _tc_pre(x, W, dinv):
    return pl.pallas_call(
        _tc_pre_body,
        out_shape=jax.ShapeDtypeStruct((N, D), jnp.float32),
    )(x, W, dinv)


def _tc_mid_body(aggp_ref, h2p_ref, dinv_ref, b_ref, g_ref, be_ref, w_ref,
                 h2n_ref):
    dinv = dinv_ref[...]
    t = (aggp_ref[0, :N, :] + aggp_ref[1, :N, :] + h2p_ref[...]) * dinv + b_ref[...]
    mu = jnp.mean(t, axis=0, keepdims=True)
    var = jnp.mean((t - mu) * (t - mu), axis=0, keepdims=True)
    y = (t - mu) * lax.rsqrt(var + 1e-5) * g_ref[...] + be_ref[...]
    y = jnp.maximum(y, 0.0)
    hw = lax.dot_general(y, w_ref[...], (((1,), (1,)), ((), ())),
                         preferred_element_type=jnp.float32)
    h2n_ref[...] = hw * dinv


def _tc_mid(aggp, h2p, dinv, b, g, be, Wn):
    return pl.pallas_call(
        _tc_mid_body,
        out_shape=jax.ShapeDtypeStruct((N, D), jnp.float32),
    )(aggp, h2p, dinv, b, g, be, Wn)


def _tc_final_body(aggp_ref, h2_ref, dinv_ref, b_ref, out_ref):
    out_ref[...] = ((aggp_ref[0, :N, :] + aggp_ref[1, :N, :] + h2_ref[...])
                    * dinv_ref[...] + b_ref[...])


def _tc_final(aggp, h2, dinv, b):
    return pl.pallas_call(
        _tc_final_body,
        out_shape=jax.ShapeDtypeStruct((N, D), jnp.float32),
    )(aggp, h2, dinv, b)


def kernel(x, edge_index, W1, b1, g1, be1, W2, b2, g2, be2, W3, b3):
    src = edge_index[0]
    dst = edge_index[1]
    npad_e = EPAD - E
    pad_ar = jnp.arange(npad_e, dtype=jnp.int32)
    pad_src = (pad_ar * 13 + 1) % N
    pad_dst = N + (pad_ar % 16)
    srcT = jnp.concatenate([src, pad_src]).reshape(NW, NCH, C)
    dstT = jnp.concatenate([dst, pad_dst]).reshape(NW, NCH, C)
    z1 = jnp.zeros((NDEG,), jnp.float32)
    zrows = jnp.zeros((NPAD, D), jnp.float32)

    degp = _sc_deg(dstT, z1)
    dinv2d = _tc_rsqrt(degp.reshape(NC, NDEG // D, D))
    dinv = dinv2d.reshape(NDEG, 1)[:N]

    b1r, g1r, be1r = b1.reshape(1, D), g1.reshape(1, D), be1.reshape(1, D)
    b2r, g2r, be2r = b2.reshape(1, D), g2.reshape(1, D), be2.reshape(1, D)
    b3r = b3.reshape(1, D)

    h2 = _tc_pre(x, W1, dinv)
    aggp = _sc_agg(h2, srcT, dstT, zrows)
    h2 = _tc_mid(aggp, h2, dinv, b1r, g1r, be1r, W2)
    aggp = _sc_agg(h2, srcT, dstT, zrows)
    h2 = _tc_mid(aggp, h2, dinv, b2r, g2r, be2r, W3)
    aggp = _sc_agg(h2, srcT, dstT, zrows)
    return _tc_final(aggp, h2, dinv, b3r)


# speedup vs baseline: 1.2730x; 1.2730x over previous
import jax
import jax.numpy as jnp
from jax import lax
from jax.experimental import pallas as pl
from jax.experimental.pallas import tpu as pltpu
from jax.experimental.pallas import tpu_sc as plsc

N = 10000
D = 128
E = 320000

NC = 2
NS = 16
NW = NC * NS
C = 128
NCH = 80
NH = NCH // 2
EPT = NCH * C
EPAD = NW * EPT
NPAD = 10112
RPT = NPAD // NS
NDEG = 10240
DPT = NDEG // NS

_mesh = plsc.VectorSubcoreMesh(core_axis_name="c", subcore_axis_name="s")


def _sc_deg_body(dstT_hbm, z1_hbm, degp_hbm, dst_v, ones_v, acc_sh):
    c = lax.axis_index("c")
    s = lax.axis_index("s")
    wid = c * NS + s
    pltpu.sync_copy(dstT_hbm.at[wid], dst_v)
    for k in range(C // 16):
        ones_v[pl.ds(k * 16, 16)] = jnp.ones((16,), jnp.float32)
    pltpu.sync_copy(z1_hbm.at[pl.ds(s * DPT, DPT)], acc_sh.at[pl.ds(s * DPT, DPT)])
    plsc.subcore_barrier()

    def step(j, carry):
        pltpu.sync_copy(ones_v, acc_sh.at[dst_v.at[j]], add=True)
        return carry

    lax.fori_loop(0, NCH, step, 0)
    plsc.subcore_barrier()
    pltpu.sync_copy(acc_sh.at[pl.ds(s * DPT, DPT)], degp_hbm.at[c, pl.ds(s * DPT, DPT)])


_sc_deg = pl.kernel(
    _sc_deg_body,
    out_type=jax.ShapeDtypeStruct((NC, NDEG), jnp.float32),
    mesh=_mesh,
    scratch_types=[
        pltpu.VMEM((NCH, C), jnp.int32),
        pltpu.VMEM((C,), jnp.float32),
        pltpu.VMEM_SHARED((NDEG,), jnp.float32),
    ],
)


def _sc_agg_body(h2_hbm, srcT_hbm, dstT_hbm, zrows_hbm, aggp_hbm,
                 src_v, dst_v, rows0, rows1, acc_sh, sg0, sg1, ss0, ss1):
    c = lax.axis_index("c")
    s = lax.axis_index("s")
    wid = c * NS + s
    pltpu.sync_copy(srcT_hbm.at[wid, pl.ds(0, NH)], src_v)
    pltpu.sync_copy(dstT_hbm.at[wid, pl.ds(0, NH)], dst_v)
    pltpu.sync_copy(zrows_hbm.at[pl.ds(s * RPT, RPT)], acc_sh.at[pl.ds(s * RPT, RPT)])
    plsc.subcore_barrier()

    for half in range(2):
        pltpu.async_copy(h2_hbm.at[src_v.at[0]], rows0, sg0)

        def step(i, carry):
            j0 = 2 * i
            j1 = j0 + 1
            j2 = jnp.minimum(j1 + 1, NH - 1)
            pltpu.async_copy(h2_hbm.at[src_v.at[j1]], rows1, sg1)
            pltpu.make_async_copy(h2_hbm.at[src_v.at[j0]], rows0, sg0).wait()
            pltpu.sync_copy(rows0, acc_sh.at[dst_v.at[j0]], add=True)
            pltpu.async_copy(h2_hbm.at[src_v.at[j2]], rows0, sg0)
            pltpu.make_async_copy(h2_hbm.at[src_v.at[j1]], rows1, sg1).wait()
            pltpu.sync_copy(rows1, acc_sh.at[dst_v.at[j1]], add=True)
            return carry

        lax.fori_loop(0, NH // 2, step, 0)
        pltpu.make_async_copy(h2_hbm.at[src_v.at[NH - 1]], rows0, sg0).wait()
        if half == 0:
            pltpu.sync_copy(srcT_hbm.at[wid, pl.ds(NH, NH)], src_v)
            pltpu.sync_copy(dstT_hbm.at[wid, pl.ds(NH, NH)], dst_v)

    plsc.subcore_barrier()
    pltpu.sync_copy(acc_sh.at[pl.ds(s * RPT, RPT)],
                    aggp_hbm.at[c, pl.ds(s * RPT, RPT)])


_sc_agg = pl.kernel(
    _sc_agg_body,
    out_type=jax.ShapeDtypeStruct((NC, NPAD, D), jnp.float32),
    mesh=_mesh,
    scratch_types=[
        pltpu.VMEM((NH, C), jnp.int32),
        pltpu.VMEM((NH, C), jnp.int32),
        pltpu.VMEM((C, D), jnp.float32),
        pltpu.VMEM((C, D), jnp.float32),
        pltpu.VMEM_SHARED((NPAD, D), jnp.float32),
        pltpu.SemaphoreType.DMA,
        pltpu.SemaphoreType.DMA,
        pltpu.SemaphoreType.DMA,
        pltpu.SemaphoreType.DMA,
    ],
)


def _tc_rsqrt_body(degp_ref, dinv_ref):
    deg = degp_ref[0] + degp_ref[1] + 1.0
    dinv_ref[...] = lax.rsqrt(deg)


def _tc_rsqrt(degp):
    return pl.pallas_call(
        _tc_rsqrt_body,
        out_shape=jax.ShapeDtypeStruct((NDEG // D, D), jnp.float32),
    )(degp)


def _tc_pre_body(x_ref, w_ref, dinv_ref, h2_ref):
    hw = lax.dot_general(x_ref[...], w_ref[...], (((1,), (1,)), ((), ())),
                         preferred_element_type=jnp.float32)
    h2_ref[...] = hw * dinv_ref[...]


def _tc_pre(x, W, dinv):
    return pl.pallas_call(
        _tc_pre_body,
        out_shape=jax.ShapeDtypeStruct((N, D), jnp.float32),
    )(x, W, dinv)


def _tc_mid_body(aggp_ref, h2p_ref, dinv_ref, b_ref, g_ref, be_ref, w_ref,
                 h2n_ref):
    dinv = dinv_ref[...]
    t = (aggp_ref[0, :N, :] + aggp_ref[1, :N, :] + h2p_ref[...]) * dinv + b_ref[...]
    mu = jnp.mean(t, axis=0, keepdims=True)
    var = jnp.mean((t - mu) * (t - mu), axis=0, keepdims=True)
    y = (t - mu) * lax.rsqrt(var + 1e-5) * g_ref[...] + be_ref[...]
    y = jnp.maximum(y, 0.0)
    hw = lax.dot_general(y, w_ref[...], (((1,), (1,)), ((), ())),
                         preferred_element_type=jnp.float32)
    h2n_ref[...] = hw * dinv


def _tc_mid(aggp, h2p, dinv, b, g, be, Wn):
    return pl.pallas_call(
        _tc_mid_body,
        out_shape=jax.ShapeDtypeStruct((N, D), jnp.float32),
    )(aggp, h2p, dinv, b, g, be, Wn)


def _tc_final_body(aggp_ref, h2_ref, dinv_ref, b_ref, out_ref):
    out_ref[...] = ((aggp_ref[0, :N, :] + aggp_ref[1, :N, :] + h2_ref[...])
                    * dinv_ref[...] + b_ref[...])


def _tc_final(aggp, h2, dinv, b):
    return pl.pallas_call(
        _tc_final_body,
        out_shape=jax.ShapeDtypeStruct((N, D), jnp.float32),
    )(aggp, h2, dinv, b)


def kernel(x, edge_index, W1, b1, g1, be1, W2, b2, g2, be2, W3, b3):
    src = edge_index[0]
    dst = edge_index[1]
    npad_e = EPAD - E
    pad_ar = jnp.arange(npad_e, dtype=jnp.int32)
    pad_src = (pad_ar * 13 + 1) % N
    pad_dst = N + (pad_ar % 16)
    srcT = jnp.concatenate([src, pad_src]).reshape(NW, NCH, C)
    dstT = jnp.concatenate([dst, pad_dst]).reshape(NW, NCH, C)
    z1 = jnp.zeros((NDEG,), jnp.float32)
    zrows = jnp.zeros((NPAD, D), jnp.float32)

    degp = _sc_deg(dstT, z1)
    dinv2d = _tc_rsqrt(degp.reshape(NC, NDEG // D, D))
    dinv = dinv2d.reshape(NDEG, 1)[:N]

    b1r, g1r, be1r = b1.reshape(1, D), g1.reshape(1, D), be1.reshape(1, D)
    b2r, g2r, be2r = b2.reshape(1, D), g2.reshape(1, D), be2.reshape(1, D)
    b3r = b3.reshape(1, D)

    h2 = _tc_pre(x, W1, dinv)
    aggp = _sc_agg(h2, srcT, dstT, zrows)
    h2 = _tc_mid(aggp, h2, dinv, b1r, g1r, be1r, W2)
    aggp = _sc_agg(h2, srcT, dstT, zrows)
    h2 = _tc_mid(aggp, h2, dinv, b2r, g2r, be2r, W3)
    aggp = _sc_agg(h2, srcT, dstT, zrows)
    return _tc_final(aggp, h2, dinv, b3r)
